# Initial kernel scaffold; baseline (speedup 1.0000x reference)
import jax
import jax.numpy as jnp
from jax.experimental import pallas as pl


def kernel(hour_idx, dow_idx, hour_table, dow_table):
    raise NotImplementedError("write your pallas kernel here")


# speedup vs baseline: 6.0398x; 6.0398x over previous
import functools

import jax
import jax.numpy as jnp
from jax import lax
from jax.experimental import pallas as pl
from jax.experimental.pallas import tpu as pltpu
from jax.experimental.pallas import tpu_sc as plsc

DIM = 128
NHOUR = 24
NDOW = 7
NCOMB = NHOUR * NDOW

NCORES = 2
NSUBCORES = 16
NW = NCORES * NSUBCORES

TOTAL = 16384 * 200
PER_W = TOTAL // NW
BLK = 4096
CH = 128
N_BLK = PER_W // BLK
N_CH = BLK // CH


def _combine_tables_tc(h_ref, d_ref, o_ref):
    o_ref[...] = h_ref[...][:, None, :] + d_ref[...][None, :, :]


_combine_tables = pl.pallas_call(
    _combine_tables_tc,
    out_shape=jax.ShapeDtypeStruct((NHOUR, NDOW, DIM), jnp.float32),
)


def _sc_lookup_body(h_hbm, d_hbm, tab_hbm, out_hbm,
                    h_v, d_v, c_v, rows_v, gsem, ssem0, ssem1):
    wid = lax.axis_index("s") * NCORES + lax.axis_index("c")
    base = wid * PER_W
    ssems = (ssem0, ssem1)

    def blk_body(b, carry):
        off = base + b * BLK
        pltpu.sync_copy(h_hbm.at[pl.ds(off, BLK)], h_v)
        pltpu.sync_copy(d_hbm.at[pl.ds(off, BLK)], d_v)

        def cbody(i, carry2):
            s = pl.ds(i * 16, 16)
            c_v[s] = h_v[s] * NDOW + d_v[s]
            return carry2

        lax.fori_loop(0, BLK // 16, cbody, 0, unroll=4)

        def gbody(p, carry2):
            for slot in range(2):
                k = 2 * p + slot
                rbuf = rows_v.at[slot]

                @pl.when(p >= 1)
                def _():
                    pltpu.make_async_copy(
                        rbuf, out_hbm.at[pl.ds(off, CH)], ssems[slot]).wait()

                pltpu.async_copy(
                    tab_hbm.at[c_v.at[pl.ds(k * CH, CH)]], rbuf, gsem).wait()
                pltpu.async_copy(
                    rbuf, out_hbm.at[pl.ds(off + k * CH, CH)], ssems[slot])
            return carry2

        lax.fori_loop(0, N_CH // 2, gbody, 0)

        for slot in range(2):
            pltpu.make_async_copy(
                rows_v.at[slot], out_hbm.at[pl.ds(off, CH)], ssems[slot]).wait()
        return carry

    lax.fori_loop(0, N_BLK, blk_body, 0)


_sc_lookup = functools.partial(
    pl.kernel,
    mesh=plsc.VectorSubcoreMesh(core_axis_name="c", subcore_axis_name="s"),
    out_type=jax.ShapeDtypeStruct((TOTAL, DIM), jnp.float32),
    scratch_types=[
        pltpu.VMEM((BLK,), jnp.int32),
        pltpu.VMEM((BLK,), jnp.int32),
        pltpu.VMEM((BLK,), jnp.int32),
        pltpu.VMEM((2, CH, DIM), jnp.float32),
        pltpu.SemaphoreType.DMA,
        pltpu.SemaphoreType.DMA,
        pltpu.SemaphoreType.DMA,
    ],
)(_sc_lookup_body)


def kernel(hour_idx, dow_idx, hour_table, dow_table):
    shape = hour_idx.shape
    h = hour_idx.reshape(-1).astype(jnp.int32)
    d = dow_idx.reshape(-1).astype(jnp.int32)
    ctable = _combine_tables(
        hour_table.astype(jnp.float32), dow_table.astype(jnp.float32)
    ).reshape(NCOMB, DIM)
    out = _sc_lookup(h, d, ctable)
    return out.reshape(shape + (DIM,))
